# baseline (speedup 1.0000x reference)
import functools

import jax
import jax.numpy as jnp
from jax import lax
from jax.experimental import pallas as pl
from jax.experimental.pallas import tpu as pltpu
from jax.experimental.pallas import tpu_sc as plsc

_NC = 2
_NS = 16
_NW = _NC * _NS

_B = 16384
_L = 200
_V = 1000

_ROWS_PER_W = _B // _NW
_HALF = _ROWS_PER_W // 2
_GROUPS_PER_HALF = _HALF // 16


def _table_body(emb_ref, w_ref, b_ref, out_ref):
    t = lax.dot_general(
        w_ref[...], emb_ref[...],
        dimension_numbers=(((1,), (1,)), ((), ())),
        preferred_element_type=jnp.float32,
    )
    t = (t + b_ref[...]) * (1.0 / _L)
    bits = lax.bitcast_convert_type(
        t.astype(jnp.bfloat16), jnp.uint16
    ).astype(jnp.uint32)
    packed = bits[0, :] | (bits[1, :] << 16)
    out_ref[...] = packed.astype(jnp.int32)


def _make_table(emb, w, b):
    return pl.pallas_call(
        _table_body,
        out_shape=jax.ShapeDtypeStruct((_V,), jnp.int32),
    )(emb, w, b.reshape(2, 1))


_sc_mesh = plsc.VectorSubcoreMesh(core_axis_name="c", subcore_axis_name="s")


@functools.partial(
    pl.kernel,
    mesh=_sc_mesh,
    out_type=(
        jax.ShapeDtypeStruct((_B,), jnp.float32),
        jax.ShapeDtypeStruct((_B,), jnp.float32),
    ),
    scratch_types=[
        pltpu.VMEM((_L, _HALF), jnp.int32),
        pltpu.VMEM((_L, _HALF), jnp.int32),
        pltpu.VMEM((_V,), jnp.int32),
        pltpu.VMEM((_ROWS_PER_W,), jnp.float32),
        pltpu.VMEM((_ROWS_PER_W,), jnp.float32),
        pltpu.SemaphoreType.DMA,
        pltpu.SemaphoreType.DMA,
    ],
    compiler_params=pltpu.CompilerParams(
        needs_layout_passes=False, use_tc_tiling_on_sc=True
    ),
)
def _sc_pool(
    t_hbm, ids_hbm, out0_hbm, out1_hbm,
    ids_v0, ids_v1, t_v, o0_v, o1_v, sem0, sem1,
):
    wid = lax.axis_index("s") * _NC + lax.axis_index("c")
    base = wid * _ROWS_PER_W
    c0 = pltpu.async_copy(ids_hbm.at[:, pl.ds(base, _HALF)], ids_v0, sem0)
    c1 = pltpu.async_copy(ids_hbm.at[:, pl.ds(base + _HALF, _HALF)], ids_v1, sem1)
    pltpu.sync_copy(t_hbm, t_v)

    n_chain = 8
    chunk_l = _L // n_chain

    c0.wait()
    for half, ids_v in ((0, ids_v0), (1, ids_v1)):
        if half == 1:
            c1.wait()

        def group_body(g, carry, ids_v=ids_v, half=half):
            col = g * 16
            z = jnp.zeros((16,), jnp.float32)

            @plsc.parallel_loop(0, chunk_l, carry=(z,) * (2 * n_chain), unroll=1)
            def accs(l, acc):
                out = []
                for k in range(n_chain):
                    idsv = ids_v[k * chunk_l + l, pl.ds(col, 16)]
                    pair = plsc.load_gather(t_v, [idsv])
                    v0, v1 = plsc.unpack(
                        plsc.bitcast(pair, jnp.bfloat16),
                        format=plsc.PackFormat.INTERLEAVED,
                    )
                    out.append(acc[2 * k] + v0)
                    out.append(acc[2 * k + 1] + v1)
                return tuple(out)

            def _tree_sum(vals):
                while len(vals) > 1:
                    vals = [
                        vals[i] + vals[i + 1] if i + 1 < len(vals) else vals[i]
                        for i in range(0, len(vals), 2)
                    ]
                return vals[0]

            a0 = _tree_sum(list(accs[0::2]))
            a1 = _tree_sum(list(accs[1::2]))
            off = half * _HALF + col
            o0_v[pl.ds(off, 16)] = a0
            o1_v[pl.ds(off, 16)] = a1
            return carry

        lax.fori_loop(0, _GROUPS_PER_HALF, group_body, 0)

    pltpu.sync_copy(o0_v, out0_hbm.at[pl.ds(base, _ROWS_PER_W)])
    pltpu.sync_copy(o1_v, out1_hbm.at[pl.ds(base, _ROWS_PER_W)])


def kernel(input_ids, emb, W, b):
    table = _make_table(emb, W, b)
    o0, o1 = _sc_pool(table, input_ids.T)
    return jnp.stack([o0, o1], axis=1)


# speedup vs baseline: 1.0123x; 1.0123x over previous
import functools

import jax
import jax.numpy as jnp
from jax import lax
from jax.experimental import pallas as pl
from jax.experimental.pallas import tpu as pltpu
from jax.experimental.pallas import tpu_sc as plsc

_NC = 2
_NS = 16
_NW = _NC * _NS

_B = 16384
_L = 200
_V = 1000

_ROWS_PER_W = _B // _NW
_HALF = _ROWS_PER_W // 2
_GROUPS_PER_HALF = _HALF // 16


def _table_body(emb_ref, w_ref, b_ref, out_ref):
    t = lax.dot_general(
        w_ref[...], emb_ref[...],
        dimension_numbers=(((1,), (1,)), ((), ())),
        preferred_element_type=jnp.float32,
    )
    t = (t + b_ref[...]) * (1.0 / _L)
    bits = lax.bitcast_convert_type(
        t.astype(jnp.bfloat16), jnp.uint16
    ).astype(jnp.uint32)
    packed = bits[0, :] | (bits[1, :] << 16)
    out_ref[...] = packed.astype(jnp.int32)


def _make_table(emb, w, b):
    return pl.pallas_call(
        _table_body,
        out_shape=jax.ShapeDtypeStruct((_V,), jnp.int32),
    )(emb, w, b.reshape(2, 1))


_sc_mesh = plsc.VectorSubcoreMesh(core_axis_name="c", subcore_axis_name="s")


@functools.partial(
    pl.kernel,
    mesh=_sc_mesh,
    out_type=(
        jax.ShapeDtypeStruct((_B,), jnp.float32),
        jax.ShapeDtypeStruct((_B,), jnp.float32),
    ),
    scratch_types=[
        pltpu.VMEM((_L, _HALF), jnp.int32),
        pltpu.VMEM((_L, _HALF), jnp.int32),
        pltpu.VMEM((_V,), jnp.int32),
        pltpu.VMEM((_ROWS_PER_W,), jnp.float32),
        pltpu.VMEM((_ROWS_PER_W,), jnp.float32),
        pltpu.SemaphoreType.DMA,
        pltpu.SemaphoreType.DMA,
    ],
    compiler_params=pltpu.CompilerParams(
        needs_layout_passes=False, use_tc_tiling_on_sc=True
    ),
)
def _sc_pool(
    t_hbm, ids_hbm, out0_hbm, out1_hbm,
    ids_v0, ids_v1, t_v, o0_v, o1_v, sem0, sem1,
):
    wid = lax.axis_index("s") * _NC + lax.axis_index("c")
    base = wid * _ROWS_PER_W
    c0 = pltpu.async_copy(ids_hbm.at[:, pl.ds(base, _HALF)], ids_v0, sem0)
    c1 = pltpu.async_copy(ids_hbm.at[:, pl.ds(base + _HALF, _HALF)], ids_v1, sem1)
    pltpu.sync_copy(t_hbm, t_v)

    n_chain = 8
    chunk_l = _L // n_chain

    c0.wait()
    for half, ids_v in ((0, ids_v0), (1, ids_v1)):
        if half == 1:
            c1.wait()

        def group_body(g, carry, ids_v=ids_v, half=half):
            col = g * 16
            z = jnp.zeros((16,), jnp.float32)

            @plsc.parallel_loop(0, chunk_l, carry=(z,) * (2 * n_chain), unroll=1)
            def accs(l, acc):
                out = []
                for k in range(n_chain):
                    idsv = ids_v[k + n_chain * l, pl.ds(col, 16)]
                    pair = plsc.load_gather(t_v, [idsv])
                    v0, v1 = plsc.unpack(
                        plsc.bitcast(pair, jnp.bfloat16),
                        format=plsc.PackFormat.INTERLEAVED,
                    )
                    out.append(acc[2 * k] + v0)
                    out.append(acc[2 * k + 1] + v1)
                return tuple(out)

            def _tree_sum(vals):
                while len(vals) > 1:
                    vals = [
                        vals[i] + vals[i + 1] if i + 1 < len(vals) else vals[i]
                        for i in range(0, len(vals), 2)
                    ]
                return vals[0]

            a0 = _tree_sum(list(accs[0::2]))
            a1 = _tree_sum(list(accs[1::2]))
            off = half * _HALF + col
            o0_v[pl.ds(off, 16)] = a0
            o1_v[pl.ds(off, 16)] = a1
            return carry

        lax.fori_loop(0, _GROUPS_PER_HALF, group_body, 0)

    pltpu.sync_copy(o0_v, out0_hbm.at[pl.ds(base, _ROWS_PER_W)])
    pltpu.sync_copy(o1_v, out1_hbm.at[pl.ds(base, _ROWS_PER_W)])


def kernel(input_ids, emb, W, b):
    table = _make_table(emb, W, b)
    o0, o1 = _sc_pool(table, input_ids.T)
    return jnp.stack([o0, o1], axis=1)


# speedup vs baseline: 1.0185x; 1.0061x over previous
import functools

import jax
import jax.numpy as jnp
from jax import lax
from jax.experimental import pallas as pl
from jax.experimental.pallas import tpu as pltpu
from jax.experimental.pallas import tpu_sc as plsc

_NC = 2
_NS = 16
_NW = _NC * _NS

_B = 16384
_L = 200
_V = 1000

_ROWS_PER_W = _B // _NW
_HALF = _ROWS_PER_W // 2
_GROUPS_PER_HALF = _HALF // 16


def _table_body(emb_ref, w_ref, b_ref, out_ref):
    t = lax.dot_general(
        w_ref[...], emb_ref[...],
        dimension_numbers=(((1,), (1,)), ((), ())),
        preferred_element_type=jnp.float32,
    )
    t = (t + b_ref[...]) * (1.0 / _L)
    bits = lax.bitcast_convert_type(
        t.astype(jnp.bfloat16), jnp.uint16
    ).astype(jnp.uint32)
    packed = bits[0, :] | (bits[1, :] << 16)
    out_ref[...] = packed.astype(jnp.int32)


def _make_table(emb, w, b):
    return pl.pallas_call(
        _table_body,
        out_shape=jax.ShapeDtypeStruct((_V,), jnp.int32),
    )(emb, w, b.reshape(2, 1))


_sc_mesh = plsc.VectorSubcoreMesh(core_axis_name="c", subcore_axis_name="s")


@functools.partial(
    pl.kernel,
    mesh=_sc_mesh,
    out_type=(
        jax.ShapeDtypeStruct((_B,), jnp.float32),
        jax.ShapeDtypeStruct((_B,), jnp.float32),
    ),
    scratch_types=[
        pltpu.VMEM((_L, _HALF), jnp.int32),
        pltpu.VMEM((_L, _HALF), jnp.int32),
        pltpu.VMEM((_V,), jnp.int32),
        pltpu.VMEM((_ROWS_PER_W,), jnp.float32),
        pltpu.VMEM((_ROWS_PER_W,), jnp.float32),
        pltpu.SemaphoreType.DMA,
        pltpu.SemaphoreType.DMA,
    ],
    compiler_params=pltpu.CompilerParams(
        needs_layout_passes=False, use_tc_tiling_on_sc=True
    ),
)
def _sc_pool(
    t_hbm, ids_hbm, out0_hbm, out1_hbm,
    ids_v0, ids_v1, t_v, o0_v, o1_v, sem0, sem1,
):
    wid = lax.axis_index("s") * _NC + lax.axis_index("c")
    base = wid * _ROWS_PER_W
    c0 = pltpu.async_copy(ids_hbm.at[:, pl.ds(base, _HALF)], ids_v0, sem0)
    c1 = pltpu.async_copy(ids_hbm.at[:, pl.ds(base + _HALF, _HALF)], ids_v1, sem1)
    pltpu.sync_copy(t_hbm, t_v)

    n_chain = 8
    chunk_l = _L // n_chain

    c0.wait()
    for half, ids_v in ((0, ids_v0), (1, ids_v1)):
        if half == 1:
            c1.wait()

        def group_body(g, carry, ids_v=ids_v, half=half):
            col = g * 16
            z = jnp.zeros((16,), jnp.float32)

            @plsc.parallel_loop(0, chunk_l, carry=(z,) * (2 * n_chain), unroll=2)
            def accs(l, acc):
                out = []
                for k in range(n_chain):
                    idsv = ids_v[k + n_chain * l, pl.ds(col, 16)]
                    pair = plsc.load_gather(t_v, [idsv])
                    v0, v1 = plsc.unpack(
                        plsc.bitcast(pair, jnp.bfloat16),
                        format=plsc.PackFormat.INTERLEAVED,
                    )
                    out.append(acc[2 * k] + v0)
                    out.append(acc[2 * k + 1] + v1)
                return tuple(out)

            def _tree_sum(vals):
                while len(vals) > 1:
                    vals = [
                        vals[i] + vals[i + 1] if i + 1 < len(vals) else vals[i]
                        for i in range(0, len(vals), 2)
                    ]
                return vals[0]

            a0 = _tree_sum(list(accs[0::2]))
            a1 = _tree_sum(list(accs[1::2]))
            off = half * _HALF + col
            o0_v[pl.ds(off, 16)] = a0
            o1_v[pl.ds(off, 16)] = a1
            return carry

        lax.fori_loop(0, _GROUPS_PER_HALF, group_body, 0)

    pltpu.sync_copy(o0_v, out0_hbm.at[pl.ds(base, _ROWS_PER_W)])
    pltpu.sync_copy(o1_v, out1_hbm.at[pl.ds(base, _ROWS_PER_W)])


def kernel(input_ids, emb, W, b):
    table = _make_table(emb, W, b)
    o0, o1 = _sc_pool(table, input_ids.T)
    return jnp.stack([o0, o1], axis=1)
